# Initial kernel scaffold; baseline (speedup 1.0000x reference)
import jax
import jax.numpy as jnp
from jax.experimental import pallas as pl


def kernel(x, gate_w, w1, w2, w3):
    raise NotImplementedError("write your pallas kernel here")


# speedup vs baseline: 2.0142x; 2.0142x over previous
import functools

import jax
import jax.numpy as jnp
from jax import lax
from jax.experimental import pallas as pl

NUM_EXPERTS = 8
TOP_K = 2
DIM = 1024
HIDDEN = 2048
TOKENS = 2048
TB = 512


def _routing_body(x_ref, gw_ref, wfull_ref):
    scores = lax.dot_general(
        x_ref[...], gw_ref[...], (((1,), (1,)), ((), ())),
        preferred_element_type=jnp.float32)
    ii = lax.broadcasted_iota(jnp.int32, scores.shape, 1)
    m1 = jnp.max(scores, axis=1, keepdims=True)
    is_m1 = scores == m1
    a1 = jnp.min(jnp.where(is_m1, ii, NUM_EXPERTS), axis=1, keepdims=True)
    oh1 = ii == a1
    masked = jnp.where(oh1, -jnp.inf, scores)
    m2 = jnp.max(masked, axis=1, keepdims=True)
    is_m2 = masked == m2
    a2 = jnp.min(jnp.where(is_m2, ii, NUM_EXPERTS), axis=1, keepdims=True)
    oh2 = ii == a2
    s = jnp.exp(m2 - m1)
    w1 = 1.0 / (1.0 + s)
    w2 = 1.0 - w1
    wfull_ref[...] = jnp.where(oh1, w1, 0.0) + jnp.where(oh2, w2, 0.0)


def _ffn_body(wfull_ref, x_ref, w1_ref, w3_ref, w2_ref, out_ref):
    e = pl.program_id(0)
    t = pl.program_id(1)
    xb = x_ref[...]
    h1 = lax.dot_general(xb, w1_ref[0], (((1,), (1,)), ((), ())),
                         preferred_element_type=jnp.float32)
    h3 = lax.dot_general(xb, w3_ref[0], (((1,), (1,)), ((), ())),
                         preferred_element_type=jnp.float32)
    g = (h1 * jax.nn.sigmoid(h1) * h3).astype(jnp.bfloat16)
    yi = lax.dot_general(g, w2_ref[0], (((1,), (1,)), ((), ())),
                         preferred_element_type=jnp.float32)
    ii = lax.broadcasted_iota(jnp.int32, (TB, NUM_EXPERTS), 1)
    wcol = jnp.sum(jnp.where(ii == e, wfull_ref[...], 0.0), axis=1,
                   keepdims=True)
    contrib = yi * wcol
    base = t * TB

    @pl.when(e == 0)
    def _():
        out_ref[pl.ds(base, TB), :] = contrib

    @pl.when(e != 0)
    def _():
        out_ref[pl.ds(base, TB), :] = out_ref[pl.ds(base, TB), :] + contrib


@jax.jit
def kernel(x, gate_w, w1, w2, w3):
    orig_shape = x.shape
    x2d = x.reshape(-1, x.shape[-1])

    wfull = pl.pallas_call(
        _routing_body,
        out_shape=jax.ShapeDtypeStruct((TOKENS, NUM_EXPERTS), jnp.float32),
    )(x2d, gate_w)

    xb = x2d.astype(jnp.bfloat16)
    w1b = w1.astype(jnp.bfloat16)
    w3b = w3.astype(jnp.bfloat16)
    w2b = w2.astype(jnp.bfloat16)

    grid = (NUM_EXPERTS, TOKENS // TB)
    y = pl.pallas_call(
        _ffn_body,
        grid=grid,
        in_specs=[
            pl.BlockSpec((TB, NUM_EXPERTS), lambda e, t: (t, 0)),
            pl.BlockSpec((TB, DIM), lambda e, t: (t, 0)),
            pl.BlockSpec((1, HIDDEN, DIM), lambda e, t: (e, 0, 0)),
            pl.BlockSpec((1, HIDDEN, DIM), lambda e, t: (e, 0, 0)),
            pl.BlockSpec((1, DIM, HIDDEN), lambda e, t: (e, 0, 0)),
        ],
        out_specs=pl.BlockSpec((TOKENS, DIM), lambda e, t: (0, 0)),
        out_shape=jax.ShapeDtypeStruct((TOKENS, DIM), jnp.float32),
    )(wfull, xb, w1b, w3b, w2b)
    return y.reshape(orig_shape)
